# baseline (speedup 1.0000x reference)
import functools

import jax
import jax.numpy as jnp
from jax import lax
from jax.experimental import pallas as pl
from jax.experimental.pallas import tpu as pltpu
from jax.experimental.pallas import tpu_sc as plsc

_MAX_T = 100000
_EMB_DIM = 16
_BATCH = 16384

_NC = 2
_NS = 16
_NW = _NC * _NS
_B_PER_W = _BATCH // _NW

_D_HALF = _EMB_DIM // 2

_mesh = plsc.VectorSubcoreMesh(core_axis_name="c", subcore_axis_name="s")


@functools.partial(
    pl.kernel,
    mesh=_mesh,
    out_type=jax.ShapeDtypeStruct((_D_HALF * _BATCH,), jnp.float32),
    scratch_types=[
        pltpu.VMEM((_B_PER_W,), jnp.int32),
        [pltpu.VMEM((_B_PER_W,), jnp.float32) for _ in range(_D_HALF)],
        [pltpu.SemaphoreType.DMA for _ in range(_D_HALF)],
        pltpu.SemaphoreType.DMA,
    ],
    compiler_params=pltpu.CompilerParams(use_tc_tiling_on_sc=False),
)
def _gather_half(flat_hbm, idx_hbm, out_hbm, idx_v, vals, sems, wsem):
    cid = lax.axis_index("c")
    sid = lax.axis_index("s")
    wid = sid * _NC + cid
    base = wid * _B_PER_W

    pltpu.sync_copy(idx_hbm.at[pl.ds(base, _B_PER_W)], idx_v)

    gathers = [
        pltpu.async_copy(
            flat_hbm.at[pl.ds(d * _MAX_T, _MAX_T)].at[idx_v],
            vals[d],
            sems[d],
        )
        for d in range(_D_HALF)
    ]
    writes = []
    for d in range(_D_HALF):
        gathers[d].wait()
        writes.append(
            pltpu.async_copy(
                vals[d], out_hbm.at[pl.ds(d * _BATCH + base, _B_PER_W)], wsem
            )
        )
    for w in writes:
        w.wait()


def kernel(t, embed_weight):
    t32 = t.astype(jnp.int32)
    table_t = embed_weight.T
    flat_a = table_t[:_D_HALF].reshape(_D_HALF * _MAX_T)
    flat_b = table_t[_D_HALF:].reshape(_D_HALF * _MAX_T)
    out_a = _gather_half(flat_a, t32).reshape(_D_HALF, _BATCH)
    out_b = _gather_half(flat_b, t32).reshape(_D_HALF, _BATCH)
    return jnp.concatenate([out_a, out_b], axis=0).T


# speedup vs baseline: 1.1698x; 1.1698x over previous
import functools

import jax
import jax.numpy as jnp
from jax import lax
from jax.experimental import pallas as pl
from jax.experimental.pallas import tpu as pltpu
from jax.experimental.pallas import tpu_sc as plsc

_MAX_T = 100000
_EMB_DIM = 16
_BATCH = 16384

_NC = 2
_NS = 16
_NW = _NC * _NS
_B_PER_W = _BATCH // _NW

_mesh = plsc.VectorSubcoreMesh(core_axis_name="c", subcore_axis_name="s")


@functools.partial(
    pl.kernel,
    mesh=_mesh,
    out_type=jax.ShapeDtypeStruct((_EMB_DIM * _BATCH,), jnp.float32),
    scratch_types=[
        pltpu.VMEM((_B_PER_W,), jnp.int32),
        pltpu.VMEM((_EMB_DIM * _B_PER_W,), jnp.float32),
        [pltpu.SemaphoreType.DMA for _ in range(_EMB_DIM)],
        pltpu.SemaphoreType.DMA,
    ],
    compiler_params=pltpu.CompilerParams(use_tc_tiling_on_sc=False),
)
def _gather_kernel(flat_hbm, idx_hbm, out_hbm, idx_v, val_all, sems, wsem):
    cid = lax.axis_index("c")
    sid = lax.axis_index("s")
    wid = sid * _NC + cid
    base = wid * _B_PER_W

    pltpu.sync_copy(idx_hbm.at[pl.ds(base, _B_PER_W)], idx_v)

    gathers = [
        pltpu.async_copy(
            flat_hbm.at[pl.ds(d * _MAX_T, _MAX_T)].at[idx_v],
            val_all.at[pl.ds(d * _B_PER_W, _B_PER_W)],
            sems[d],
        )
        for d in range(_EMB_DIM)
    ]
    for g in gathers:
        g.wait()
    pltpu.async_copy(
        val_all,
        out_hbm.at[pl.ds(wid * _EMB_DIM * _B_PER_W, _EMB_DIM * _B_PER_W)],
        wsem,
    ).wait()


def kernel(t, embed_weight):
    flat = embed_weight.T.reshape(_EMB_DIM * _MAX_T)
    out_w = _gather_kernel(flat, t.astype(jnp.int32))
    out_t = out_w.reshape(_NW, _EMB_DIM, _B_PER_W).transpose(1, 0, 2)
    return out_t.reshape(_EMB_DIM, _BATCH).T


# speedup vs baseline: 1.1731x; 1.0029x over previous
import functools

import jax
import jax.numpy as jnp
from jax import lax
from jax.experimental import pallas as pl
from jax.experimental.pallas import tpu as pltpu
from jax.experimental.pallas import tpu_sc as plsc

_MAX_T = 100000
_EMB_DIM = 16
_BATCH = 16384

_NC = 2
_NS = 16
_NW = _NC * _NS
_B_PER_W = _BATCH // _NW

_mesh = plsc.VectorSubcoreMesh(core_axis_name="c", subcore_axis_name="s")


@functools.partial(
    pl.kernel,
    mesh=_mesh,
    out_type=jax.ShapeDtypeStruct((_EMB_DIM * _BATCH,), jnp.float32),
    scratch_types=[
        pltpu.VMEM((_B_PER_W,), jnp.int32),
        pltpu.VMEM((_EMB_DIM * _B_PER_W,), jnp.float32),
        [pltpu.SemaphoreType.DMA for _ in range(_EMB_DIM)],
        pltpu.SemaphoreType.DMA,
    ],
    compiler_params=pltpu.CompilerParams(use_tc_tiling_on_sc=False),
)
def _gather_kernel(flat_hbm, idx_hbm, out_hbm, idx_v, val_all, sems, wsem):
    cid = lax.axis_index("c")
    sid = lax.axis_index("s")
    wid = sid * _NC + cid
    base = wid * _B_PER_W

    pltpu.sync_copy(idx_hbm.at[pl.ds(base, _B_PER_W)], idx_v)

    gathers = [
        pltpu.async_copy(
            flat_hbm.at[pl.ds(d * _MAX_T, _MAX_T)].at[idx_v],
            val_all.at[pl.ds(d * _B_PER_W, _B_PER_W)],
            sems[d],
        )
        for d in range(_EMB_DIM)
    ]
    half = _EMB_DIM // 2 * _B_PER_W
    for g in gathers[: _EMB_DIM // 2]:
        g.wait()
    w0 = pltpu.async_copy(
        val_all.at[pl.ds(0, half)],
        out_hbm.at[pl.ds(wid * _EMB_DIM * _B_PER_W, half)],
        wsem,
    )
    for g in gathers[_EMB_DIM // 2 :]:
        g.wait()
    w1 = pltpu.async_copy(
        val_all.at[pl.ds(half, half)],
        out_hbm.at[pl.ds(wid * _EMB_DIM * _B_PER_W + half, half)],
        wsem,
    )
    w0.wait()
    w1.wait()


def kernel(t, embed_weight):
    flat = embed_weight.T.reshape(_EMB_DIM * _MAX_T)
    out_w = _gather_kernel(flat, t.astype(jnp.int32))
    out_t = out_w.reshape(_NW, _EMB_DIM, _B_PER_W).transpose(1, 0, 2)
    return out_t.reshape(_EMB_DIM, _BATCH).T
